# baseline (speedup 1.0000x reference)
import jax
import jax.numpy as jnp
from jax.experimental import pallas as pl
from jax.experimental.pallas import tpu as pltpu

_B, _SEQ, _C, _H = 32, 4096, 192, 4
_W = _SEQ // _H
_FIX = 384
_OVL = 4
_NVQ = 6
_K = 1024
_VD = 256
_COMMIT = 0.25
_NROW = _B * _W // _OVL
_RB = 512


def _down_kernel(z_ref, wd_ref, out_ref):
    acc = jnp.zeros((_W, _FIX), jnp.float32)
    for h in range(_H):
        acc = acc + jnp.dot(z_ref[0, h], wd_ref[:, h * _FIX:(h + 1) * _FIX],
                            preferred_element_type=jnp.float32)
    out_ref[0] = acc


def _vq_kernel(zo_ref, e_ref, zq_ref, loss_ref, eb_ref):
    @pl.when(pl.program_id(0) == 0)
    def _cast():
        eb_ref[...] = e_ref[...].astype(jnp.bfloat16)

    total = jnp.float32(0.0)
    for i in range(_NVQ):
        zf = zo_ref[:, i * _VD:(i + 1) * _VD]
        ei = e_ref[i]
        z2 = jnp.sum(zf * zf, axis=1, keepdims=True)
        e2 = jnp.sum(ei * ei, axis=0, keepdims=True)
        dist = z2 - 2.0 * jnp.dot(zf, ei, preferred_element_type=jnp.float32) + e2
        idx = jnp.argmin(dist, axis=1)
        total = total + jnp.sum(jnp.min(dist, axis=1))
        onehot = (jax.lax.broadcasted_iota(jnp.int32, (_RB, _K), 1)
                  == idx[:, None]).astype(jnp.bfloat16)
        zq = jax.lax.dot_general(onehot, eb_ref[i],
                                 (((1,), (1,)), ((), ())),
                                 preferred_element_type=jnp.float32)
        zq_ref[:, i * _VD:(i + 1) * _VD] = zq.astype(jnp.bfloat16)

    @pl.when(pl.program_id(0) == 0)
    def _init():
        loss_ref[...] = jnp.full((8, 128), total, jnp.float32)

    @pl.when(pl.program_id(0) != 0)
    def _acc():
        loss_ref[...] = loss_ref[...] + jnp.full((8, 128), total, jnp.float32)


def _up_kernel(zq_ref, wu_ref, out_ref, wut_ref):
    @pl.when(pl.program_id(0) == 0)
    def _build():
        rows = jax.lax.broadcasted_iota(jnp.int32, (_C * _H, _C), 0)
        cols = jax.lax.broadcasted_iota(jnp.int32, (_C * _H, _C), 1)
        for h in range(_H):
            sel = (rows == cols * _H + h).astype(jnp.float32)
            wut_ref[h] = jnp.dot(wu_ref[...], sel,
                                 preferred_element_type=jnp.float32
                                 ).astype(jnp.bfloat16)

    x = zq_ref[0]
    for h in range(_H):
        out_ref[0, h] = jnp.dot(x, wut_ref[h], preferred_element_type=jnp.float32)


def kernel(z, Wd, Wu, E):
    z4 = z.reshape(_B, _H, _W, _C)
    wd2 = Wd.reshape(_C, _H * _FIX)
    zp = pl.pallas_call(
        _down_kernel,
        grid=(_B,),
        in_specs=[pl.BlockSpec((1, _H, _W, _C), lambda b: (b, 0, 0, 0)),
                  pl.BlockSpec((_C, _H * _FIX), lambda b: (0, 0))],
        out_specs=pl.BlockSpec((1, _W, _FIX), lambda b: (b, 0, 0)),
        out_shape=jax.ShapeDtypeStruct((_B, _W, _FIX), jnp.float32),
    )(z4, wd2)

    zo = zp.reshape(_NROW, _OVL * _FIX)
    nblk = _NROW // _RB
    zq, lossb = pl.pallas_call(
        _vq_kernel,
        grid=(nblk,),
        in_specs=[pl.BlockSpec((_RB, _OVL * _FIX), lambda r: (r, 0)),
                  pl.BlockSpec((_NVQ, _VD, _K), lambda r: (0, 0, 0))],
        out_specs=[pl.BlockSpec((_RB, _OVL * _FIX), lambda r: (r, 0)),
                   pl.BlockSpec((8, 128), lambda r: (0, 0))],
        out_shape=[jax.ShapeDtypeStruct((_NROW, _OVL * _FIX), jnp.bfloat16),
                   jax.ShapeDtypeStruct((8, 128), jnp.float32)],
        scratch_shapes=[pltpu.VMEM((_NVQ, _VD, _K), jnp.bfloat16)],
    )(zo, E)

    zqp = zq.reshape(_B, _W, _FIX)
    out = pl.pallas_call(
        _up_kernel,
        grid=(_B,),
        in_specs=[pl.BlockSpec((1, _W, _FIX), lambda b: (b, 0, 0)),
                  pl.BlockSpec((_FIX, _C * _H), lambda b: (0, 0))],
        out_specs=pl.BlockSpec((1, _H, _W, _C), lambda b: (b, 0, 0, 0)),
        out_shape=jax.ShapeDtypeStruct((_B, _H, _W, _C), jnp.float32),
        scratch_shapes=[pltpu.VMEM((_H, _FIX, _C), jnp.bfloat16)],
    )(zqp, Wu)

    zq_out = out.reshape(_B, _SEQ, _C)
    loss = lossb[0, 0] * (_COMMIT / (_NROW * _VD * _NVQ))
    return zq_out, loss


# speedup vs baseline: 1.2367x; 1.2367x over previous
import jax
import jax.numpy as jnp
from jax.experimental import pallas as pl
from jax.experimental.pallas import tpu as pltpu

_B, _SEQ, _C, _H = 32, 4096, 192, 4
_W = _SEQ // _H
_FIX = 384
_OVL = 4
_NVQ = 6
_K = 1024
_VD = 256
_COMMIT = 0.25
_NROW = _B * _W // _OVL
_JB = _W // _OVL


def _fused_kernel(z_ref, wd_ref, wu_ref, e_ref, out_ref, loss_ref,
                  eb_ref, wut_ref):
    @pl.when(pl.program_id(0) == 0)
    def _prep():
        eb_ref[...] = e_ref[...].astype(jnp.bfloat16)
        rows = jax.lax.broadcasted_iota(jnp.int32, (_C * _H, _C), 0)
        cols = jax.lax.broadcasted_iota(jnp.int32, (_C * _H, _C), 1)
        for h in range(_H):
            sel = (rows == cols * _H + h).astype(jnp.float32)
            wut_ref[h] = jnp.dot(wu_ref[...], sel,
                                 preferred_element_type=jnp.float32
                                 ).astype(jnp.bfloat16)

    zp = jnp.zeros((_W, _FIX), jnp.float32)
    for h in range(_H):
        zp = zp + jnp.dot(z_ref[0, h], wd_ref[:, h * _FIX:(h + 1) * _FIX],
                          preferred_element_type=jnp.float32)
    zo = zp.reshape(_JB, _OVL * _FIX)

    total = jnp.float32(0.0)
    zq_cols = []
    for i in range(_NVQ):
        zf = zo[:, i * _VD:(i + 1) * _VD]
        ei = e_ref[i]
        z2 = jnp.sum(zf * zf, axis=1, keepdims=True)
        e2 = jnp.sum(ei * ei, axis=0, keepdims=True)
        dist = z2 - 2.0 * jnp.dot(zf, ei, preferred_element_type=jnp.float32) + e2
        idx = jnp.argmin(dist, axis=1)
        total = total + jnp.sum(jnp.min(dist, axis=1))
        onehot = (jax.lax.broadcasted_iota(jnp.int32, (_JB, _K), 1)
                  == idx[:, None]).astype(jnp.bfloat16)
        zq_cols.append(jax.lax.dot_general(
            onehot, eb_ref[i], (((1,), (1,)), ((), ())),
            preferred_element_type=jnp.float32).astype(jnp.bfloat16))

    zq = jnp.concatenate(zq_cols, axis=1)
    zqp = zq.reshape(_W, _FIX)

    for h in range(_H):
        out_ref[0, h] = jnp.dot(zqp, wut_ref[h],
                                preferred_element_type=jnp.float32)

    @pl.when(pl.program_id(0) == 0)
    def _init():
        loss_ref[...] = jnp.full((8, 128), total, jnp.float32)

    @pl.when(pl.program_id(0) != 0)
    def _acc():
        loss_ref[...] = loss_ref[...] + jnp.full((8, 128), total, jnp.float32)


def kernel(z, Wd, Wu, E):
    z4 = z.reshape(_B, _H, _W, _C)
    wd2 = Wd.reshape(_C, _H * _FIX)
    out, lossb = pl.pallas_call(
        _fused_kernel,
        grid=(_B,),
        in_specs=[pl.BlockSpec((1, _H, _W, _C), lambda b: (b, 0, 0, 0)),
                  pl.BlockSpec((_C, _H * _FIX), lambda b: (0, 0)),
                  pl.BlockSpec((_FIX, _C * _H), lambda b: (0, 0)),
                  pl.BlockSpec((_NVQ, _VD, _K), lambda b: (0, 0, 0))],
        out_specs=[pl.BlockSpec((1, _H, _W, _C), lambda b: (b, 0, 0, 0)),
                   pl.BlockSpec((8, 128), lambda b: (0, 0))],
        out_shape=[jax.ShapeDtypeStruct((_B, _H, _W, _C), jnp.float32),
                   jax.ShapeDtypeStruct((8, 128), jnp.float32)],
        scratch_shapes=[pltpu.VMEM((_NVQ, _VD, _K), jnp.bfloat16),
                        pltpu.VMEM((_H, _FIX, _C), jnp.bfloat16)],
    )(z4, wd2, Wu, E)

    zq_out = out.reshape(_B, _SEQ, _C)
    loss = lossb[0, 0] * (_COMMIT / (_NROW * _VD * _NVQ))
    return zq_out, loss
